# baseline (speedup 1.0000x reference)
import functools

import jax
import jax.numpy as jnp
from jax import lax
from jax.experimental import pallas as pl
from jax.experimental.pallas import tpu as pltpu
from jax.experimental.pallas import tpu_sc as plsc

_B, _N, _K, _H, _W = 4, 16384, 128, 256, 256
_HW = _H * _W
_IMG_H = 256
_NC, _NS, _L = 2, 16, 16
_NW = _NC * _NS
_WPB = _NW // _B
_KPW = _K // _WPB
_SENT = _HW
_PAD = 272
_PCH = 2048
_OCH = 4096
_WSCALE = 16384.0
_WINV = 1.0 / 16384.0


def _sc_body(fi, coords, out, img, idxb, wqb, pxb, pyb, obufa, obufb, sema, semb):
    cid = lax.axis_index("c")
    sid = lax.axis_index("s")
    wid = sid * _NC + cid
    b = wid // _WPB
    kbase = (wid % _WPB) * _KPW

    zeros = jnp.zeros((_L,), jnp.float32)
    for r in (_H, _H + 1):
        for g in range(_W // _L):
            img[r, pl.ds(g * _L, _L)] = zeros

    for c in range(_N // _PCH):
        base = c * _PCH
        pltpu.sync_copy(coords.at[b, 0, pl.ds(base, _PCH)], pxb)
        pltpu.sync_copy(coords.at[b, 1, pl.ds(base, _PCH)], pyb)

        @plsc.parallel_loop(0, _PCH // _L, unroll=4)
        def proj(g):
            o = g * _L
            x = pxb[pl.ds(o, _L)]
            y = pyb[pl.ds(o, _L)]
            valid = (x >= 0.0) & (y >= 0.0) & (x < float(_IMG_H)) & (y < float(_IMG_H))
            xc = jnp.minimum(jnp.maximum(x, 1.0), float(_IMG_H - 2))
            yc = jnp.minimum(jnp.maximum(y, 1.0), float(_IMG_H - 2))
            xi = xc.astype(jnp.int32)
            yi = yc.astype(jnp.int32)
            wx = xc - xi.astype(jnp.float32)
            wy = yc - yi.astype(jnp.float32)
            idx = yi * _W + xi
            wxq = (wx * _WSCALE + 0.5).astype(jnp.int32)
            wyq = (wy * _WSCALE + 0.5).astype(jnp.int32)
            wq = lax.shift_left(wxq, 16) + wyq
            idxb[pl.ds(base + o, _L)] = jnp.where(valid, idx, _SENT)
            wqb[pl.ds(base + o, _L)] = jnp.where(valid, wq, 0)

        del proj

    def chan(t, carry):
        kk = kbase + t
        row = b * _K + kk
        pltpu.sync_copy(fi.at[b, kk], img.at[pl.ds(0, _H)])

        def compute_chunk(oc_static, obuf):
            ob = oc_static * _OCH

            @plsc.parallel_loop(0, _OCH // _L, unroll=8)
            def interp(g):
                o = g * _L
                i0 = idxb[pl.ds(ob + o, _L)]
                wq = wqb[pl.ds(ob + o, _L)]
                wx = lax.shift_right_logical(wq, 16).astype(jnp.float32) * _WINV
                wy = lax.bitwise_and(wq, 65535).astype(jnp.float32) * _WINV
                iy = lax.shift_right_logical(i0, 8)
                ix = lax.bitwise_and(i0, 255)
                iy1 = iy + 1
                ix1 = ix + 1
                f00 = plsc.load_gather(img, [iy, ix])
                f01 = plsc.load_gather(img, [iy, ix1])
                f10 = plsc.load_gather(img, [iy1, ix])
                f11 = plsc.load_gather(img, [iy1, ix1])
                wx1 = 1.0 - wx
                a = f00 * wx1 + f01 * wx
                bb = f10 * wx1 + f11 * wx
                obuf[pl.ds(o, _L)] = a * (1.0 - wy) + bb * wy

            del interp

        compute_chunk(0, obufa)
        cpa = pltpu.async_copy(obufa, out.at[row, pl.ds(0 * _OCH, _OCH)], sema)
        compute_chunk(1, obufb)
        cpb = pltpu.async_copy(obufb, out.at[row, pl.ds(1 * _OCH, _OCH)], semb)
        cpa.wait()
        compute_chunk(2, obufa)
        cpa = pltpu.async_copy(obufa, out.at[row, pl.ds(2 * _OCH, _OCH)], sema)
        cpb.wait()
        compute_chunk(3, obufb)
        cpb = pltpu.async_copy(obufb, out.at[row, pl.ds(3 * _OCH, _OCH)], semb)
        cpa.wait()
        cpb.wait()
        return carry

    lax.fori_loop(0, _KPW, chan, 0)


@functools.partial(
    pl.kernel,
    out_type=jax.ShapeDtypeStruct((_B * _K, _N), jnp.float32),
    mesh=plsc.VectorSubcoreMesh(core_axis_name="c", subcore_axis_name="s"),
    compiler_params=pltpu.CompilerParams(needs_layout_passes=False),
    scratch_types=[
        pltpu.VMEM((_H + 2, _W), jnp.float32),
        pltpu.VMEM((_N,), jnp.int32),
        pltpu.VMEM((_N,), jnp.int32),
        pltpu.VMEM((_PCH,), jnp.float32),
        pltpu.VMEM((_PCH,), jnp.float32),
        pltpu.VMEM((_OCH,), jnp.float32),
        pltpu.VMEM((_OCH,), jnp.float32),
        pltpu.SemaphoreType.DMA,
        pltpu.SemaphoreType.DMA,
    ],
)
def _sc_interp(fi, coords, out, img, idxb, wqb, pxb, pyb, obufa, obufb, sema, semb):
    _sc_body(fi, coords, out, img, idxb, wqb, pxb, pyb, obufa, obufb, sema, semb)


def kernel(point_set, feature_image, extrinsics, intrinsics):
    ps = jnp.concatenate([point_set, jnp.ones_like(point_set[:, :, 0:1])], axis=-1)
    ps_homog = jnp.transpose(ps, (0, 2, 1))
    cam_points = (jnp.linalg.inv(extrinsics).astype(jnp.float32) @ ps_homog)[:, :3]
    im_coords = intrinsics @ cam_points
    im_coords_homog = (im_coords / im_coords[:, -1:, :])[:, :2, :]

    out = _sc_interp(feature_image, im_coords_homog)
    return out.reshape(_B, _K, _N)


# speedup vs baseline: 1.0121x; 1.0121x over previous
import functools

import jax
import jax.numpy as jnp
from jax import lax
from jax.experimental import pallas as pl
from jax.experimental.pallas import tpu as pltpu
from jax.experimental.pallas import tpu_sc as plsc

_B, _N, _K, _H, _W = 4, 16384, 128, 256, 256
_HW = _H * _W
_IMG_H = 256
_NC, _NS, _L = 2, 16, 16
_NW = _NC * _NS
_WPB = _NW // _B
_KPW = _K // _WPB
_SENT = _HW
_PAD = 272
_PCH = 2048
_OCH = 4096
_WSCALE = 16384.0
_WINV = 1.0 / 16384.0


def _sc_body(fi, coords, out, img, idxb, wqb, pxb, pyb, obufa, obufb, sema, semb):
    cid = lax.axis_index("c")
    sid = lax.axis_index("s")
    wid = sid * _NC + cid
    b = wid // _WPB
    kbase = (wid % _WPB) * _KPW

    zeros = jnp.zeros((_L,), jnp.float32)
    for r in (_H, _H + 1):
        for g in range(_W // _L):
            img[r, pl.ds(g * _L, _L)] = zeros

    for c in range(_N // _PCH):
        base = c * _PCH
        pltpu.sync_copy(coords.at[b, 0, pl.ds(base, _PCH)], pxb)
        pltpu.sync_copy(coords.at[b, 1, pl.ds(base, _PCH)], pyb)

        @plsc.parallel_loop(0, _PCH // _L, unroll=4)
        def proj(g):
            o = g * _L
            x = pxb[pl.ds(o, _L)]
            y = pyb[pl.ds(o, _L)]
            valid = (x >= 0.0) & (y >= 0.0) & (x < float(_IMG_H)) & (y < float(_IMG_H))
            xc = jnp.minimum(jnp.maximum(x, 1.0), float(_IMG_H - 2))
            yc = jnp.minimum(jnp.maximum(y, 1.0), float(_IMG_H - 2))
            xi = xc.astype(jnp.int32)
            yi = yc.astype(jnp.int32)
            wx = xc - xi.astype(jnp.float32)
            wy = yc - yi.astype(jnp.float32)
            idx = yi * _W + xi
            wxq = (wx * _WSCALE + 0.5).astype(jnp.int32)
            wyq = (wy * _WSCALE + 0.5).astype(jnp.int32)
            wq = lax.shift_left(wxq, 16) + wyq
            idxb[pl.ds(base + o, _L)] = jnp.where(valid, idx, _SENT)
            wqb[pl.ds(base + o, _L)] = jnp.where(valid, wq, 0)

        del proj

    def chan(t, carry):
        kk = kbase + t
        row = b * _K + kk
        pltpu.sync_copy(fi.at[b, kk], img.at[pl.ds(0, _H)])

        def compute_chunk(oc_static, obuf):
            ob = oc_static * _OCH

            @plsc.parallel_loop(0, _OCH // _L, unroll=4)
            def interp(g):
                o = g * _L
                i0 = idxb[pl.ds(ob + o, _L)]
                wq = wqb[pl.ds(ob + o, _L)]
                wx = lax.shift_right_logical(wq, 16).astype(jnp.float32) * _WINV
                wy = lax.bitwise_and(wq, 65535).astype(jnp.float32) * _WINV
                iy = lax.shift_right_logical(i0, 8)
                ix = lax.bitwise_and(i0, 255)
                iy1 = iy + 1
                ix1 = ix + 1
                f00 = plsc.load_gather(img, [iy, ix])
                f01 = plsc.load_gather(img, [iy, ix1])
                f10 = plsc.load_gather(img, [iy1, ix])
                f11 = plsc.load_gather(img, [iy1, ix1])
                wx1 = 1.0 - wx
                a = f00 * wx1 + f01 * wx
                bb = f10 * wx1 + f11 * wx
                obuf[pl.ds(o, _L)] = a * (1.0 - wy) + bb * wy

            del interp

        compute_chunk(0, obufa)
        cpa = pltpu.async_copy(obufa, out.at[row, pl.ds(0 * _OCH, _OCH)], sema)
        compute_chunk(1, obufb)
        cpb = pltpu.async_copy(obufb, out.at[row, pl.ds(1 * _OCH, _OCH)], semb)
        cpa.wait()
        compute_chunk(2, obufa)
        cpa = pltpu.async_copy(obufa, out.at[row, pl.ds(2 * _OCH, _OCH)], sema)
        cpb.wait()
        compute_chunk(3, obufb)
        cpb = pltpu.async_copy(obufb, out.at[row, pl.ds(3 * _OCH, _OCH)], semb)
        cpa.wait()
        cpb.wait()
        return carry

    lax.fori_loop(0, _KPW, chan, 0)


@functools.partial(
    pl.kernel,
    out_type=jax.ShapeDtypeStruct((_B * _K, _N), jnp.float32),
    mesh=plsc.VectorSubcoreMesh(core_axis_name="c", subcore_axis_name="s"),
    compiler_params=pltpu.CompilerParams(needs_layout_passes=False),
    scratch_types=[
        pltpu.VMEM((_H + 2, _W), jnp.float32),
        pltpu.VMEM((_N,), jnp.int32),
        pltpu.VMEM((_N,), jnp.int32),
        pltpu.VMEM((_PCH,), jnp.float32),
        pltpu.VMEM((_PCH,), jnp.float32),
        pltpu.VMEM((_OCH,), jnp.float32),
        pltpu.VMEM((_OCH,), jnp.float32),
        pltpu.SemaphoreType.DMA,
        pltpu.SemaphoreType.DMA,
    ],
)
def _sc_interp(fi, coords, out, img, idxb, wqb, pxb, pyb, obufa, obufb, sema, semb):
    _sc_body(fi, coords, out, img, idxb, wqb, pxb, pyb, obufa, obufb, sema, semb)


def kernel(point_set, feature_image, extrinsics, intrinsics):
    ps = jnp.concatenate([point_set, jnp.ones_like(point_set[:, :, 0:1])], axis=-1)
    ps_homog = jnp.transpose(ps, (0, 2, 1))
    cam_points = (jnp.linalg.inv(extrinsics).astype(jnp.float32) @ ps_homog)[:, :3]
    im_coords = intrinsics @ cam_points
    im_coords_homog = (im_coords / im_coords[:, -1:, :])[:, :2, :]

    out = _sc_interp(feature_image, im_coords_homog)
    return out.reshape(_B, _K, _N)


# speedup vs baseline: 1.1131x; 1.0998x over previous
import functools

import jax
import jax.numpy as jnp
from jax import lax
from jax.experimental import pallas as pl
from jax.experimental.pallas import tpu as pltpu
from jax.experimental.pallas import tpu_sc as plsc

_B, _N, _K, _H, _W = 4, 16384, 128, 256, 256
_HW = _H * _W
_IMG_H = 256
_NC, _NS, _L = 2, 16, 16
_NW = _NC * _NS
_WPB = _NW // _B
_KPW = _K // _WPB
_SENT = _HW
_PAD = 272
_PCH = 2048
_OCH = 4096


def _sc_body(fi, coords, out, img, idxb, wxb, wyb, pxb, pyb, obufa, obufb, sema, semb, semi):
    cid = lax.axis_index("c")
    sid = lax.axis_index("s")
    wid = sid * _NC + cid
    b = wid // _WPB
    kbase = (wid % _WPB) * _KPW

    zeros = jnp.zeros((_L,), jnp.float32)
    for r in (_H, _H + 1):
        for g in range(_W // _L):
            img[r, pl.ds(g * _L, _L)] = zeros

    pltpu.async_copy(fi.at[b, kbase], img.at[pl.ds(0, _H)], semi)

    for c in range(_N // _PCH):
        base = c * _PCH
        pltpu.sync_copy(coords.at[b, 0, pl.ds(base, _PCH)], pxb)
        pltpu.sync_copy(coords.at[b, 1, pl.ds(base, _PCH)], pyb)

        @plsc.parallel_loop(0, _PCH // _L, unroll=4)
        def proj(g):
            o = g * _L
            x = pxb[pl.ds(o, _L)]
            y = pyb[pl.ds(o, _L)]
            valid = (x >= 0.0) & (y >= 0.0) & (x < float(_IMG_H)) & (y < float(_IMG_H))
            xc = jnp.minimum(jnp.maximum(x, 1.0), float(_IMG_H - 2))
            yc = jnp.minimum(jnp.maximum(y, 1.0), float(_IMG_H - 2))
            xi = xc.astype(jnp.int32)
            yi = yc.astype(jnp.int32)
            wx = xc - xi.astype(jnp.float32)
            wy = yc - yi.astype(jnp.float32)
            idx = yi * _W + xi
            idxb[pl.ds(base + o, _L)] = jnp.where(valid, idx, _SENT)
            wxb[pl.ds(base + o, _L)] = jnp.where(valid, wx, 0.0)
            wyb[pl.ds(base + o, _L)] = jnp.where(valid, wy, 0.0)

        del proj

    def chan(t, carry):
        kk = kbase + t
        row = b * _K + kk
        pltpu.make_async_copy(fi.at[b, kk], img.at[pl.ds(0, _H)], semi).wait()

        def compute_chunk(oc_static, obuf):
            ob = oc_static * _OCH

            @plsc.parallel_loop(0, _OCH // _L, unroll=4)
            def interp(g):
                o = g * _L
                i0 = idxb[pl.ds(ob + o, _L)]
                wx = wxb[pl.ds(ob + o, _L)]
                wy = wyb[pl.ds(ob + o, _L)]
                iy = lax.shift_right_logical(i0, 8)
                ix = lax.bitwise_and(i0, 255)
                iy1 = iy + 1
                ix1 = ix + 1
                f00 = plsc.load_gather(img, [iy, ix])
                f01 = plsc.load_gather(img, [iy, ix1])
                f10 = plsc.load_gather(img, [iy1, ix])
                f11 = plsc.load_gather(img, [iy1, ix1])
                wx1 = 1.0 - wx
                a = f00 * wx1 + f01 * wx
                bb = f10 * wx1 + f11 * wx
                obuf[pl.ds(o, _L)] = a * (1.0 - wy) + bb * wy

            del interp

        compute_chunk(0, obufa)
        cpa = pltpu.async_copy(obufa, out.at[row, pl.ds(0 * _OCH, _OCH)], sema)
        compute_chunk(1, obufb)
        cpb = pltpu.async_copy(obufb, out.at[row, pl.ds(1 * _OCH, _OCH)], semb)
        cpa.wait()
        compute_chunk(2, obufa)
        cpa = pltpu.async_copy(obufa, out.at[row, pl.ds(2 * _OCH, _OCH)], sema)
        cpb.wait()
        compute_chunk(3, obufb)
        cpb = pltpu.async_copy(obufb, out.at[row, pl.ds(3 * _OCH, _OCH)], semb)
        kn = kbase + jnp.minimum(t + 1, _KPW - 1)
        pltpu.async_copy(fi.at[b, kn], img.at[pl.ds(0, _H)], semi)
        cpa.wait()
        cpb.wait()
        return carry

    lax.fori_loop(0, _KPW, chan, 0)
    pltpu.make_async_copy(fi.at[b, kbase], img.at[pl.ds(0, _H)], semi).wait()


@functools.partial(
    pl.kernel,
    out_type=jax.ShapeDtypeStruct((_B * _K, _N), jnp.float32),
    mesh=plsc.VectorSubcoreMesh(core_axis_name="c", subcore_axis_name="s"),
    compiler_params=pltpu.CompilerParams(needs_layout_passes=False),
    scratch_types=[
        pltpu.VMEM((_H + 2, _W), jnp.float32),
        pltpu.VMEM((_N,), jnp.int32),
        pltpu.VMEM((_N,), jnp.float32),
        pltpu.VMEM((_N,), jnp.float32),
        pltpu.VMEM((_PCH,), jnp.float32),
        pltpu.VMEM((_PCH,), jnp.float32),
        pltpu.VMEM((_OCH,), jnp.float32),
        pltpu.VMEM((_OCH,), jnp.float32),
        pltpu.SemaphoreType.DMA,
        pltpu.SemaphoreType.DMA,
        pltpu.SemaphoreType.DMA,
    ],
)
def _sc_interp(fi, coords, out, img, idxb, wxb, wyb, pxb, pyb, obufa, obufb, sema, semb, semi):
    _sc_body(fi, coords, out, img, idxb, wxb, wyb, pxb, pyb, obufa, obufb, sema, semb, semi)


def kernel(point_set, feature_image, extrinsics, intrinsics):
    ps = jnp.concatenate([point_set, jnp.ones_like(point_set[:, :, 0:1])], axis=-1)
    ps_homog = jnp.transpose(ps, (0, 2, 1))
    cam_points = (jnp.linalg.inv(extrinsics).astype(jnp.float32) @ ps_homog)[:, :3]
    im_coords = intrinsics @ cam_points
    im_coords_homog = (im_coords / im_coords[:, -1:, :])[:, :2, :]

    out = _sc_interp(feature_image, im_coords_homog)
    return out.reshape(_B, _K, _N)


# speedup vs baseline: 1.1179x; 1.0043x over previous
import functools

import jax
import jax.numpy as jnp
from jax import lax
from jax.experimental import pallas as pl
from jax.experimental.pallas import tpu as pltpu
from jax.experimental.pallas import tpu_sc as plsc

_B, _N, _K, _H, _W = 4, 16384, 128, 256, 256
_HW = _H * _W
_IMG_H = 256
_NC, _NS, _L = 2, 16, 16
_NW = _NC * _NS
_WPB = _NW // _B
_KPW = _K // _WPB
_SENT = _HW
_PAD = 272
_PCH = 2048
_OCH = 4096


def _sc_body(fi, coords, out, img, idxb, wxb, wyb, pxb, pyb, obufa, obufb, sema, semb, semi):
    cid = lax.axis_index("c")
    sid = lax.axis_index("s")
    wid = sid * _NC + cid
    b = wid // _WPB
    kbase = (wid % _WPB) * _KPW

    zeros = jnp.zeros((_L,), jnp.float32)
    for r in (_H, _H + 1):
        for g in range(_W // _L):
            img[r, pl.ds(g * _L, _L)] = zeros

    pltpu.async_copy(fi.at[b, kbase], img.at[pl.ds(0, _H)], semi)

    for c in range(_N // _PCH):
        base = c * _PCH
        pltpu.sync_copy(coords.at[b, 0, pl.ds(base, _PCH)], pxb)
        pltpu.sync_copy(coords.at[b, 1, pl.ds(base, _PCH)], pyb)

        @plsc.parallel_loop(0, _PCH // _L, unroll=4)
        def proj(g):
            o = g * _L
            x = pxb[pl.ds(o, _L)]
            y = pyb[pl.ds(o, _L)]
            valid = (x >= 0.0) & (y >= 0.0) & (x < float(_IMG_H)) & (y < float(_IMG_H))
            xc = jnp.minimum(jnp.maximum(x, 1.0), float(_IMG_H - 2))
            yc = jnp.minimum(jnp.maximum(y, 1.0), float(_IMG_H - 2))
            xi = xc.astype(jnp.int32)
            yi = yc.astype(jnp.int32)
            wx = xc - xi.astype(jnp.float32)
            wy = yc - yi.astype(jnp.float32)
            idx = yi * _W + xi
            idxb[pl.ds(base + o, _L)] = jnp.where(valid, idx, _SENT)
            wxb[pl.ds(base + o, _L)] = jnp.where(valid, wx, 0.0)
            wyb[pl.ds(base + o, _L)] = jnp.where(valid, wy, 0.0)

        del proj

    def chan(t, carry):
        kk = kbase + t
        row = b * _K + kk
        pltpu.make_async_copy(fi.at[b, kk], img.at[pl.ds(0, _H)], semi).wait()

        def compute_chunk(oc_static, obuf):
            ob = oc_static * _OCH

            @plsc.parallel_loop(0, _OCH // _L, unroll=4)
            def interp(g):
                o = g * _L
                i0 = idxb[pl.ds(ob + o, _L)]
                wx = wxb[pl.ds(ob + o, _L)]
                wy = wyb[pl.ds(ob + o, _L)]
                iy = lax.shift_right_logical(i0, 8)
                ix = lax.bitwise_and(i0, 255)
                iy1 = iy + 1
                ix1 = ix + 1
                f00 = plsc.load_gather(img, [iy, ix])
                f01 = plsc.load_gather(img, [iy, ix1])
                f10 = plsc.load_gather(img, [iy1, ix])
                f11 = plsc.load_gather(img, [iy1, ix1])
                a = f00 + (f01 - f00) * wx
                bb = f10 + (f11 - f10) * wx
                obuf[pl.ds(o, _L)] = a + (bb - a) * wy

            del interp

        compute_chunk(0, obufa)
        cpa = pltpu.async_copy(obufa, out.at[row, pl.ds(0 * _OCH, _OCH)], sema)
        compute_chunk(1, obufb)
        cpb = pltpu.async_copy(obufb, out.at[row, pl.ds(1 * _OCH, _OCH)], semb)
        cpa.wait()
        compute_chunk(2, obufa)
        cpa = pltpu.async_copy(obufa, out.at[row, pl.ds(2 * _OCH, _OCH)], sema)
        cpb.wait()
        compute_chunk(3, obufb)
        cpb = pltpu.async_copy(obufb, out.at[row, pl.ds(3 * _OCH, _OCH)], semb)
        kn = kbase + jnp.minimum(t + 1, _KPW - 1)
        pltpu.async_copy(fi.at[b, kn], img.at[pl.ds(0, _H)], semi)
        cpa.wait()
        cpb.wait()
        return carry

    lax.fori_loop(0, _KPW, chan, 0)
    pltpu.make_async_copy(fi.at[b, kbase], img.at[pl.ds(0, _H)], semi).wait()


@functools.partial(
    pl.kernel,
    out_type=jax.ShapeDtypeStruct((_B * _K, _N), jnp.float32),
    mesh=plsc.VectorSubcoreMesh(core_axis_name="c", subcore_axis_name="s"),
    compiler_params=pltpu.CompilerParams(needs_layout_passes=False),
    scratch_types=[
        pltpu.VMEM((_H + 2, _W), jnp.float32),
        pltpu.VMEM((_N,), jnp.int32),
        pltpu.VMEM((_N,), jnp.float32),
        pltpu.VMEM((_N,), jnp.float32),
        pltpu.VMEM((_PCH,), jnp.float32),
        pltpu.VMEM((_PCH,), jnp.float32),
        pltpu.VMEM((_OCH,), jnp.float32),
        pltpu.VMEM((_OCH,), jnp.float32),
        pltpu.SemaphoreType.DMA,
        pltpu.SemaphoreType.DMA,
        pltpu.SemaphoreType.DMA,
    ],
)
def _sc_interp(fi, coords, out, img, idxb, wxb, wyb, pxb, pyb, obufa, obufb, sema, semb, semi):
    _sc_body(fi, coords, out, img, idxb, wxb, wyb, pxb, pyb, obufa, obufb, sema, semb, semi)


def kernel(point_set, feature_image, extrinsics, intrinsics):
    ps = jnp.concatenate([point_set, jnp.ones_like(point_set[:, :, 0:1])], axis=-1)
    ps_homog = jnp.transpose(ps, (0, 2, 1))
    cam_points = (jnp.linalg.inv(extrinsics).astype(jnp.float32) @ ps_homog)[:, :3]
    im_coords = intrinsics @ cam_points
    im_coords_homog = (im_coords / im_coords[:, -1:, :])[:, :2, :]

    out = _sc_interp(feature_image, im_coords_homog)
    return out.reshape(_B, _K, _N)
